# Initial kernel scaffold; baseline (speedup 1.0000x reference)
import jax
import jax.numpy as jnp
from jax.experimental import pallas as pl


def kernel(expanded_x, expanded_row_idx, x1, x2, bias, scales, expert_idx, drop_pad_mode):
    raise NotImplementedError("write your pallas kernel here")


# speedup vs baseline: 3.4809x; 3.4809x over previous
import jax
import jax.numpy as jnp
from jax import lax
from jax.experimental import pallas as pl
from jax.experimental.pallas import tpu as pltpu
from jax.experimental.pallas import tpu_sc as plsc

N = 16384
K = 2
H = 1024
E = 16
NW = 32
RPW = N // NW
C = 8
NCHUNK = RPW // C
L = 16
NSL = H // L


def _body(ex_hbm, idx0_hbm, idx1_hbm, x1_hbm, x2_hbm, biasf_hbm,
          s0_hbm, s1_hbm, e0_hbm, e1_hbm, out_hbm,
          idx0_v, idx1_v, s0_v, s1_v, e0_v, e1_v, bias_v,
          g0_v, g1_v, x2_v, out_v,
          in_sems, out_sems):
  wid = lax.axis_index("s") * 2 + lax.axis_index("c")
  base = wid * RPW

  pltpu.sync_copy(idx0_hbm.at[pl.ds(base, RPW)], idx0_v)
  pltpu.sync_copy(idx1_hbm.at[pl.ds(base, RPW)], idx1_v)
  pltpu.sync_copy(s0_hbm.at[pl.ds(base, RPW)], s0_v)
  pltpu.sync_copy(s1_hbm.at[pl.ds(base, RPW)], s1_v)
  pltpu.sync_copy(e0_hbm.at[pl.ds(base, RPW)], e0_v)
  pltpu.sync_copy(e1_hbm.at[pl.ds(base, RPW)], e1_v)
  pltpu.sync_copy(biasf_hbm, bias_v)

  def issue_in(b2, b4, c):
    rb = base + c * C
    pltpu.async_copy(ex_hbm.at[idx0_v.at[pl.ds(c * C, C)]], g0_v.at[b2],
                     in_sems[b2])
    pltpu.async_copy(ex_hbm.at[idx1_v.at[pl.ds(c * C, C)]], g1_v.at[b2],
                     in_sems[b2])
    pltpu.async_copy(x1_hbm.at[pl.ds(rb, C)], out_v.at[b4], in_sems[b2])
    pltpu.async_copy(x2_hbm.at[pl.ds(rb, C)], x2_v.at[b2], in_sems[b2])

  def wait_in(b2, b4, c):
    pltpu.make_async_copy(ex_hbm.at[idx0_v.at[pl.ds(c * C, C)]], g0_v.at[b2],
                          in_sems[b2]).wait()
    pltpu.make_async_copy(ex_hbm.at[idx1_v.at[pl.ds(c * C, C)]], g1_v.at[b2],
                          in_sems[b2]).wait()
    rb = base + c * C
    pltpu.make_async_copy(x1_hbm.at[pl.ds(rb, C)], out_v.at[b4],
                          in_sems[b2]).wait()
    pltpu.make_async_copy(x2_hbm.at[pl.ds(rb, C)], x2_v.at[b2],
                          in_sems[b2]).wait()

  def issue_out(b4, c):
    rb = base + c * C
    pltpu.async_copy(out_v.at[b4], out_hbm.at[pl.ds(rb, C)], out_sems[b4])

  def wait_out(b4, c):
    rb = base + c * C
    pltpu.make_async_copy(out_v.at[b4], out_hbm.at[pl.ds(rb, C)],
                          out_sems[b4]).wait()

  lane = lax.iota(jnp.int32, L)

  def compute(b2, b4, c):
    s0rs, s1rs, eb0s, eb1s = [], [], [], []
    for r in range(C):
      rowvec = jnp.full((L,), c * C + r, jnp.int32)
      s0rs.append(plsc.load_gather(s0_v, [rowvec]))
      s1rs.append(plsc.load_gather(s1_v, [rowvec]))
      eb0s.append(plsc.load_gather(e0_v, [rowvec]) * H + lane)
      eb1s.append(plsc.load_gather(e1_v, [rowvec]) * H + lane)

    @plsc.parallel_loop(0, NSL, step=1, unroll=1)
    def _(h):
      off = h * L
      for r in range(C):
        g0 = g0_v[b2, r, pl.ds(off, L)]
        g1 = g1_v[b2, r, pl.ds(off, L)]
        a2 = x2_v[b2, r, pl.ds(off, L)]
        b0 = plsc.load_gather(bias_v, [eb0s[r] + off])
        b1 = plsc.load_gather(bias_v, [eb1s[r] + off])
        val = a2 + s0rs[r] * (g0 + b0) + s1rs[r] * (g1 + b1)
        plsc.addupdate(out_v.at[b4, r, pl.ds(off, L)], val)

  def do_chunk(b2, b4, c):
    wait_in(b2, b4, c)
    compute(b2, b4, c)
    issue_out(b4, c)

    @pl.when(c + 2 < NCHUNK)
    def _():
      @pl.when(c >= 2)
      def _():
        wait_out((b4 + 2) % 4, c - 2)
      issue_in(b2, (b4 + 2) % 4, c + 2)

  issue_in(0, 0, jnp.int32(0))
  issue_in(1, 1, jnp.int32(1))

  def step(j, _):
    for k in range(4):
      do_chunk(k % 2, k, 4 * j + k)
    return 0

  lax.fori_loop(0, NCHUNK // 4, step, 0)
  for c in range(NCHUNK - 4, NCHUNK):
    wait_out(c % 4, jnp.int32(c))


@jax.jit
def _run(ex, idx0, idx1, x1, x2, biasf, s0, s1, e0, e1):
  mesh = plsc.VectorSubcoreMesh(core_axis_name="c", subcore_axis_name="s")
  f = pl.kernel(
      _body,
      out_type=jax.ShapeDtypeStruct((N, H), jnp.float32),
      mesh=mesh,
      compiler_params=pltpu.CompilerParams(needs_layout_passes=False),
      scratch_types=[
          pltpu.VMEM((RPW,), jnp.int32),
          pltpu.VMEM((RPW,), jnp.int32),
          pltpu.VMEM((RPW,), jnp.float32),
          pltpu.VMEM((RPW,), jnp.float32),
          pltpu.VMEM((RPW,), jnp.int32),
          pltpu.VMEM((RPW,), jnp.int32),
          pltpu.VMEM((E * H,), jnp.float32),
          pltpu.VMEM((2, C, H), jnp.float32),
          pltpu.VMEM((2, C, H), jnp.float32),
          pltpu.VMEM((2, C, H), jnp.float32),
          pltpu.VMEM((4, C, H), jnp.float32),
          [pltpu.SemaphoreType.DMA] * 2,
          [pltpu.SemaphoreType.DMA] * 4,
      ],
  )
  return f(ex, idx0, idx1, x1, x2, biasf, s0, s1, e0, e1)


def kernel(expanded_x, expanded_row_idx, x1, x2, bias, scales, expert_idx,
           drop_pad_mode=0):
  idx0 = expanded_row_idx[:N]
  idx1 = expanded_row_idx[N:]
  biasf = bias.reshape(E * H)
  s0 = scales[:, 0]
  s1 = scales[:, 1]
  e0 = expert_idx[:, 0]
  e1 = expert_idx[:, 1]
  return _run(expanded_x, idx0, idx1, x1, x2, biasf, s0, s1, e0, e1)
